# Initial kernel scaffold; baseline (speedup 1.0000x reference)
import jax
import jax.numpy as jnp
from jax.experimental import pallas as pl


def kernel(rois, roi_cls_loc, roi_scores):
    raise NotImplementedError("write your pallas kernel here")


# speedup vs baseline: 79.1980x; 79.1980x over previous
import functools

import jax
import jax.numpy as jnp
from jax import lax
from jax.experimental import pallas as pl
from jax.experimental.pallas import tpu as pltpu

N_CLASSES = 21
IMG_H, IMG_W = 600.0, 800.0
NMS_THRESH = 0.3
SCORE_THRESH = 0.05
LOC_STD = (0.1, 0.1, 0.2, 0.2)
BIG_I32 = 2**30


def _decode_body(rois_ref, loc_ref, scores_ref, s0_ref, s1_ref, s2_ref, s3_ref,
                 bbox_ref, prob_ref, boxout_ref):
    rois = rois_ref[:, :]
    loc = loc_ref[:, :]
    scores = scores_ref[:, :]
    npad, ncols = loc.shape

    col = lax.broadcasted_iota(jnp.int32, (npad, ncols), 1)
    k = col % 4
    std = jnp.where(k < 2, 0.1, 0.2)
    loc = loc * std

    src_y0 = rois[:, 0:1]
    src_x0 = rois[:, 1:2]
    h = rois[:, 2:3] - src_y0
    w = rois[:, 3:4] - src_x0
    cy = src_y0 + 0.5 * h
    cx = src_x0 + 0.5 * w

    dy_s = jnp.dot(loc, s0_ref[:, :], preferred_element_type=jnp.float32)
    dx_s = jnp.dot(loc, s1_ref[:, :], preferred_element_type=jnp.float32)
    dh_s = jnp.dot(loc, s2_ref[:, :], preferred_element_type=jnp.float32)
    dw_s = jnp.dot(loc, s3_ref[:, :], preferred_element_type=jnp.float32)

    ct_y = dy_s * h + cy
    ct_x = dx_s * w + cx
    hh = jnp.exp(dh_s) * h
    ww = jnp.exp(dw_s) * w

    is_y = (k % 2) == 0
    ct = jnp.where(is_y, ct_y, ct_x)
    sz = jnp.where(is_y, hh, ww)
    sign = jnp.where(k < 2, -0.5, 0.5)
    out = ct + sign * sz
    out = jnp.clip(out, 0.0, jnp.where(is_y, IMG_H, IMG_W))
    bbox_ref[:, :] = out

    m = jnp.max(scores, axis=1, keepdims=True)
    e = jnp.exp(scores - m)
    prob_ref[:, :] = e / jnp.sum(e, axis=1, keepdims=True)

    b1 = out[:, 4:8]
    area = (b1[:, 2:3] - b1[:, 0:1]) * (b1[:, 3:4] - b1[:, 1:2])
    boxout_ref[:, :] = jnp.concatenate(
        [b1, area, jnp.zeros((npad, 3), jnp.float32)], axis=1)


def _nms_body(probt_ref, boxd_ref, out_ref):
    probt = probt_ref[:, :]
    nc, npad = probt.shape
    y0 = boxd_ref[0:1, :]
    x0 = boxd_ref[1:2, :]
    y1 = boxd_ref[2:3, :]
    x1 = boxd_ref[3:4, :]
    ar = boxd_ref[4:5, :]

    iota = lax.broadcasted_iota(jnp.int32, (nc, npad), 1)
    work0 = jnp.where(probt > SCORE_THRESH, probt, -1.0)
    masked0 = jnp.zeros((nc, npad), jnp.float32)

    def cond(c):
        work, _ = c
        return jnp.max(work) > 0.0

    def body(c):
        work, masked = c
        m = jnp.max(work, axis=1, keepdims=True)
        active = m > 0.0
        sel = jnp.min(jnp.where(work == m, iota, BIG_I32), axis=1,
                      keepdims=True)
        onehot = (iota == sel) & active
        y0s = jnp.sum(jnp.where(onehot, y0, 0.0), axis=1, keepdims=True)
        x0s = jnp.sum(jnp.where(onehot, x0, 0.0), axis=1, keepdims=True)
        y1s = jnp.sum(jnp.where(onehot, y1, 0.0), axis=1, keepdims=True)
        x1s = jnp.sum(jnp.where(onehot, x1, 0.0), axis=1, keepdims=True)
        ars = jnp.sum(jnp.where(onehot, ar, 0.0), axis=1, keepdims=True)
        hh = jnp.maximum(jnp.minimum(y1, y1s) - jnp.maximum(y0, y0s), 0.0)
        ww = jnp.maximum(jnp.minimum(x1, x1s) - jnp.maximum(x0, x0s), 0.0)
        inter = hh * ww
        sup = (1.3 * inter > 0.3 * (ar + ars + 1e-9)) & active
        masked = masked + jnp.where(onehot, probt, 0.0)
        work = jnp.where(sup | onehot, -1.0, work)
        return work, masked

    _, masked = lax.while_loop(cond, body, (work0, masked0))
    out_ref[:, :] = masked


def _spread_mats(ncols):
    import numpy as np
    cols = np.arange(ncols)
    mats = []
    for kk in range(4):
        s = np.zeros((ncols, ncols), np.float32)
        s[4 * (cols // 4) + kk, cols] = 1.0
        mats.append(jnp.asarray(s))
    return mats


@jax.jit
def kernel(rois, roi_cls_loc, roi_scores):
    n = rois.shape[0]
    npad = ((n + 15) // 16) * 16
    pad = npad - n
    rois_p = jnp.pad(rois, ((0, pad), (0, 0)))
    loc_p = jnp.pad(roi_cls_loc, ((0, pad), (0, 0)))
    scores_p = jnp.pad(roi_scores, ((0, pad), (0, 0)))
    ncols = roi_cls_loc.shape[1]
    nc = roi_scores.shape[1]

    s0, s1, s2, s3 = _spread_mats(ncols)
    bbox, prob, boxout = pl.pallas_call(
        _decode_body,
        out_shape=[
            jax.ShapeDtypeStruct((npad, ncols), jnp.float32),
            jax.ShapeDtypeStruct((npad, nc), jnp.float32),
            jax.ShapeDtypeStruct((npad, 8), jnp.float32),
        ],
    )(rois_p, loc_p, scores_p, s0, s1, s2, s3)

    probt = prob[:, 1:].T
    boxd = boxout.T

    masked_t = pl.pallas_call(
        _nms_body,
        out_shape=jax.ShapeDtypeStruct((nc - 1, npad), jnp.float32),
    )(probt, boxd)

    return jnp.concatenate([bbox[:n], masked_t.T[:n]], axis=1)
